# baseline (speedup 1.0000x reference)
import jax
import jax.numpy as jnp
from jax.experimental import pallas as pl
from jax.experimental.pallas import tpu as pltpu

VOCAB = 8192
LANES = 128
SUBROWS = VOCAB // LANES
ROWS_PER_STEP = 16


def _body(x_ref, t_ref, *refs):
    tbl_refs = refs[:ROWS_PER_STEP]
    out_ref, loss_ref = refs[ROWS_PER_STEP], refs[ROWS_PER_STEP + 1]
    i = pl.program_id(0)
    nsteps = pl.num_programs(0)

    shape = (1, SUBROWS, LANES)
    col = (
        jax.lax.broadcasted_iota(jnp.int32, shape, 1) * LANES
        + jax.lax.broadcasted_iota(jnp.int32, shape, 2)
    )
    nll = 0.0
    for j in range(ROWS_PER_STEP):
        b = tbl_refs[j][...]
        out_ref[j : j + 1] = b
        m = jnp.max(b)
        s = jnp.sum(jnp.exp(b - m))
        tv = t_ref[i * ROWS_PER_STEP + j]
        tg = jnp.sum(jnp.where(col == tv, b, 0.0))
        nll = nll + (m + jnp.log(s) - tg)

    nll_sum = jnp.full((1, 1), nll, jnp.float32)
    prev = jnp.where(i == 0, jnp.zeros((1, 1), jnp.float32), loss_ref[...])
    tot = prev + nll_sum
    n_tokens = nsteps * ROWS_PER_STEP
    loss_ref[...] = jnp.where(i == nsteps - 1, tot / n_tokens, tot)


def kernel(x, targets, table):
    B, T = x.shape
    n = B * T
    xf = x.reshape(-1)
    tf = targets.reshape(-1)
    tbl3 = table.reshape(VOCAB, SUBROWS, LANES)
    nsteps = n // ROWS_PER_STEP

    grid_spec = pltpu.PrefetchScalarGridSpec(
        num_scalar_prefetch=2,
        grid=(nsteps,),
        in_specs=[
            pl.BlockSpec(
                (1, SUBROWS, LANES),
                lambda i, xr, tr, j=j: (xr[i * ROWS_PER_STEP + j], 0, 0),
            )
            for j in range(ROWS_PER_STEP)
        ],
        out_specs=[
            pl.BlockSpec(
                (ROWS_PER_STEP, SUBROWS, LANES), lambda i, xr, tr: (i, 0, 0)
            ),
            pl.BlockSpec((1, 1), lambda i, xr, tr: (0, 0)),
        ],
    )

    logits3d, loss2d = pl.pallas_call(
        _body,
        grid_spec=grid_spec,
        out_shape=[
            jax.ShapeDtypeStruct((n, SUBROWS, LANES), jnp.float32),
            jax.ShapeDtypeStruct((1, 1), jnp.float32),
        ],
        compiler_params=pltpu.CompilerParams(
            dimension_semantics=("arbitrary",),
        ),
    )(xf, tf, *([tbl3] * ROWS_PER_STEP))

    return logits3d.reshape(B, T, VOCAB), loss2d[0, 0]


# speedup vs baseline: 2.0392x; 2.0392x over previous
import jax
import jax.numpy as jnp
from jax.experimental import pallas as pl
from jax.experimental.pallas import tpu as pltpu

VOCAB = 8192
LANES = 128
SUBROWS = VOCAB // LANES
ROWS_PER_STEP = 16


def _body(x_ref, t_ref, *refs):
    tbl_refs = refs[:ROWS_PER_STEP]
    out_ref, loss_ref = refs[ROWS_PER_STEP], refs[ROWS_PER_STEP + 1]
    i = pl.program_id(0)
    nsteps = pl.num_programs(0)

    blocks = []
    for j in range(ROWS_PER_STEP):
        b = tbl_refs[j][...]
        out_ref[j : j + 1] = b
        blocks.append(b)
    block = jnp.concatenate(blocks, axis=0)

    m = jnp.max(block, axis=(1, 2), keepdims=True)
    s = jnp.sum(jnp.exp(block - m), axis=(1, 2), keepdims=True)

    tv = jnp.stack([t_ref[i * ROWS_PER_STEP + j] for j in range(ROWS_PER_STEP)])
    shape = (ROWS_PER_STEP, SUBROWS, LANES)
    col = (
        jax.lax.broadcasted_iota(jnp.int32, shape, 1) * LANES
        + jax.lax.broadcasted_iota(jnp.int32, shape, 2)
    )
    tgt = jnp.sum(
        jnp.where(col == tv[:, None, None], block, 0.0),
        axis=(1, 2),
        keepdims=True,
    )

    nll_sum = jnp.sum(m + jnp.log(s) - tgt).reshape(1, 1)
    prev = jnp.where(i == 0, jnp.zeros((1, 1), jnp.float32), loss_ref[...])
    tot = prev + nll_sum
    n_tokens = nsteps * ROWS_PER_STEP
    loss_ref[...] = jnp.where(i == nsteps - 1, tot / n_tokens, tot)


def kernel(x, targets, table):
    B, T = x.shape
    n = B * T
    xf = x.reshape(-1)
    tf = targets.reshape(-1)
    tbl3 = table.reshape(VOCAB, SUBROWS, LANES)
    nsteps = n // ROWS_PER_STEP

    grid_spec = pltpu.PrefetchScalarGridSpec(
        num_scalar_prefetch=2,
        grid=(nsteps,),
        in_specs=[
            pl.BlockSpec(
                (1, SUBROWS, LANES),
                lambda i, xr, tr, j=j: (xr[i * ROWS_PER_STEP + j], 0, 0),
            )
            for j in range(ROWS_PER_STEP)
        ],
        out_specs=[
            pl.BlockSpec(
                (ROWS_PER_STEP, SUBROWS, LANES), lambda i, xr, tr: (i, 0, 0)
            ),
            pl.BlockSpec((1, 1), lambda i, xr, tr: (0, 0)),
        ],
    )

    logits3d, loss2d = pl.pallas_call(
        _body,
        grid_spec=grid_spec,
        out_shape=[
            jax.ShapeDtypeStruct((n, SUBROWS, LANES), jnp.float32),
            jax.ShapeDtypeStruct((1, 1), jnp.float32),
        ],
        compiler_params=pltpu.CompilerParams(
            dimension_semantics=("arbitrary",),
        ),
    )(xf, tf, *([tbl3] * ROWS_PER_STEP))

    return logits3d.reshape(B, T, VOCAB), loss2d[0, 0]
